# baseline (speedup 1.0000x reference)
import jax
import jax.numpy as jnp
from jax.experimental import pallas as pl
from jax.experimental.pallas import tpu as pltpu

_N = 10000
_D = 128
_BM = 400
_BC = 200
_NB = _N // _BM
_STEPS = 2 * _NB
_NBUF = 3


def _g1_kernel(x_ref, w1_ref, b1_ref, o_ref):
    acc = jnp.dot(x_ref[...], w1_ref[...],
                  preferred_element_type=jnp.float32) + b1_ref[...]
    o_ref[...] = acc.astype(jnp.bfloat16)


def _seq_row(seq):
    return jnp.where(seq < _NB, seq, seq - _NB)


def _fetch(adj_ref, abuf, sems, seq):
    slot = jax.lax.rem(seq, _NBUF)
    row = _seq_row(seq) * _BM
    return pltpu.make_async_copy(
        adj_ref.at[pl.ds(row, _BM), :], abuf.at[slot], sems.at[slot])


def _stream_kernel(g1_ref, adj_ref, w2_ref, b2_ref,
                   w3_ref, b3_ref, w4_ref, b4_ref,
                   emb_ref, z_ref, g2_ref, abuf, sems):
    i = pl.program_id(0)

    @pl.when(i == 0)
    def _prologue():
        for t in range(_NBUF - 1):
            _fetch(adj_ref, abuf, sems, t).start()

    @pl.when(i + _NBUF - 1 < _STEPS)
    def _lookahead():
        _fetch(adj_ref, abuf, sems, i + _NBUF - 1).start()

    _fetch(adj_ref, abuf, sems, i).wait()
    slot = jax.lax.rem(i, _NBUF)

    @pl.when(i < _NB)
    def _pass1():
        for c in range(_BM // _BC):
            a = abuf[slot, c * _BC:(c + 1) * _BC, :].astype(jnp.bfloat16)
            h = jnp.dot(a, g1_ref[...], preferred_element_type=jnp.float32)
            h = jnp.maximum(h, 0.0)
            g2 = jnp.dot(h, w2_ref[...],
                         preferred_element_type=jnp.float32) + b2_ref[...]
            g2_ref[pl.ds(i * _BM + c * _BC, _BC), :] = g2.astype(jnp.bfloat16)

    @pl.when(i >= _NB)
    def _pass2():
        for c in range(_BM // _BC):
            a = abuf[slot, c * _BC:(c + 1) * _BC, :].astype(jnp.bfloat16)
            emb = jnp.dot(a, g2_ref[...], preferred_element_type=jnp.float32)
            emb_ref[c * _BC:(c + 1) * _BC, :] = emb
            t = jnp.dot(emb, w3_ref[...],
                        preferred_element_type=jnp.float32) + b3_ref[...]
            t = jnp.maximum(t, 0.0)
            z_ref[c * _BC:(c + 1) * _BC, :] = jnp.dot(
                t, w4_ref[...], preferred_element_type=jnp.float32) + b4_ref[...]


def _out_map(i):
    return (jnp.clip(i - _NB, 0, _NB - 1), 0)


def _const_map(i):
    return (0, 0)


def kernel(x, Adj_, W1, b1, W2, b2, W3, b3, W4, b4):
    full = lambda r, c: pl.BlockSpec((r, c), _const_map)

    g1 = pl.pallas_call(
        _g1_kernel,
        out_shape=jax.ShapeDtypeStruct((_N, _D), jnp.bfloat16),
    )(x, W1, b1.reshape(1, _D))

    emb, z = pl.pallas_call(
        _stream_kernel,
        grid=(_STEPS,),
        in_specs=[
            full(_N, _D),
            pl.BlockSpec(memory_space=pltpu.MemorySpace.HBM),
            full(_D, _D), full(1, _D),
            full(_D, _D), full(1, _D),
            full(_D, _D), full(1, _D),
        ],
        out_specs=[
            pl.BlockSpec((_BM, _D), _out_map),
            pl.BlockSpec((_BM, _D), _out_map),
        ],
        out_shape=[
            jax.ShapeDtypeStruct((_N, _D), jnp.float32),
            jax.ShapeDtypeStruct((_N, _D), jnp.float32),
        ],
        scratch_shapes=[
            pltpu.VMEM((_N, _D), jnp.bfloat16),
            pltpu.VMEM((_NBUF, _BM, _N), jnp.float32),
            pltpu.SemaphoreType.DMA((_NBUF,)),
        ],
    )(g1, Adj_, W2, b2.reshape(1, _D), W3, b3.reshape(1, _D),
      W4, b4.reshape(1, _D))
    return (z, emb)


# speedup vs baseline: 1.0004x; 1.0004x over previous
import jax
import jax.numpy as jnp
from jax.experimental import pallas as pl
from jax.experimental.pallas import tpu as pltpu

_N = 10000
_D = 128
_BM = 400
_BC = 200
_NB = _N // _BM


def _fused_kernel(x_ref, adj_ref, w1_ref, b1_ref, w2_ref, b2_ref,
                  w3_ref, b3_ref, w4_ref, b4_ref,
                  emb_ref, z_ref, g1_ref, g2_ref):
    i = pl.program_id(0)

    @pl.when(i == 0)
    def _g1_phase():
        acc = jnp.dot(x_ref[...], w1_ref[...].astype(jnp.bfloat16),
                      preferred_element_type=jnp.float32) + b1_ref[...]
        g1_ref[...] = acc.astype(jnp.bfloat16)

    @pl.when((i >= 1) & (i <= _NB))
    def _pass1_phase():
        for c in range(_BM // _BC):
            a = adj_ref[c * _BC:(c + 1) * _BC, :].astype(jnp.bfloat16)
            h = jnp.dot(a, g1_ref[...], preferred_element_type=jnp.float32)
            h = jnp.maximum(h, 0.0)
            g2 = jnp.dot(h, w2_ref[...],
                         preferred_element_type=jnp.float32) + b2_ref[...]
            g2_ref[pl.ds((i - 1) * _BM + c * _BC, _BC), :] = (
                g2.astype(jnp.bfloat16))

    @pl.when(i > _NB)
    def _pass2_phase():
        for c in range(_BM // _BC):
            a = adj_ref[c * _BC:(c + 1) * _BC, :].astype(jnp.bfloat16)
            emb = jnp.dot(a, g2_ref[...], preferred_element_type=jnp.float32)
            emb_ref[c * _BC:(c + 1) * _BC, :] = emb
            t = jnp.dot(emb, w3_ref[...],
                        preferred_element_type=jnp.float32) + b3_ref[...]
            t = jnp.maximum(t, 0.0)
            z_ref[c * _BC:(c + 1) * _BC, :] = jnp.dot(
                t, w4_ref[...], preferred_element_type=jnp.float32) + b4_ref[...]


def _adj_map(i):
    return (jnp.where(i <= _NB, jnp.maximum(i - 1, 0), i - 1 - _NB), 0)


def _out_map(i):
    return (jnp.clip(i - 1 - _NB, 0, _NB - 1), 0)


def _const_map(i):
    return (0, 0)


def kernel(x, Adj_, W1, b1, W2, b2, W3, b3, W4, b4):
    full = lambda r, c: pl.BlockSpec((r, c), _const_map)
    emb, z = pl.pallas_call(
        _fused_kernel,
        grid=(1 + 2 * _NB,),
        in_specs=[
            full(_N, _D),
            pl.BlockSpec((_BM, _N), _adj_map),
            full(_D, _D), full(1, _D),
            full(_D, _D), full(1, _D),
            full(_D, _D), full(1, _D),
            full(_D, _D), full(1, _D),
        ],
        out_specs=[
            pl.BlockSpec((_BM, _D), _out_map),
            pl.BlockSpec((_BM, _D), _out_map),
        ],
        out_shape=[
            jax.ShapeDtypeStruct((_N, _D), jnp.float32),
            jax.ShapeDtypeStruct((_N, _D), jnp.float32),
        ],
        scratch_shapes=[
            pltpu.VMEM((_N, _D), jnp.bfloat16),
            pltpu.VMEM((_N, _D), jnp.bfloat16),
        ],
        compiler_params=pltpu.CompilerParams(
            vmem_limit_bytes=64 * 1024 * 1024,
        ),
    )(x.astype(jnp.bfloat16), Adj_, W1, b1.reshape(1, _D),
      W2, b2.reshape(1, _D), W3, b3.reshape(1, _D), W4, b4.reshape(1, _D))
    return (z, emb)


# speedup vs baseline: 1.0258x; 1.0254x over previous
import jax
import jax.numpy as jnp
from jax.experimental import pallas as pl
from jax.experimental.pallas import tpu as pltpu

_N = 10000
_D = 128
_BM = 400
_NB = _N // _BM


def _fused_kernel(x_ref, adj_ref, w1_ref, b1_ref, w2_ref, b2_ref,
                  w3_ref, b3_ref, w4_ref, b4_ref,
                  emb_ref, z_ref, g1_ref, g2_ref):
    i = pl.program_id(0)

    @pl.when(i == 0)
    def _g1_phase():
        acc = jnp.dot(x_ref[...], w1_ref[...],
                      preferred_element_type=jnp.float32) + b1_ref[...]
        g1_ref[...] = acc.astype(jnp.bfloat16)

    @pl.when((i >= 1) & (i <= _NB))
    def _pass1_phase():
        a = adj_ref[...].astype(jnp.bfloat16)
        h = jnp.dot(a, g1_ref[...], preferred_element_type=jnp.float32)
        h = jnp.maximum(h, 0.0)
        g2 = jnp.dot(h, w2_ref[...],
                     preferred_element_type=jnp.float32) + b2_ref[...]
        g2_ref[pl.ds((i - 1) * _BM, _BM), :] = g2.astype(jnp.bfloat16)

    @pl.when(i > _NB)
    def _pass2_phase():
        a = adj_ref[...].astype(jnp.bfloat16)
        emb = jnp.dot(a, g2_ref[...], preferred_element_type=jnp.float32)
        emb_ref[...] = emb
        t = jnp.dot(emb, w3_ref[...],
                    preferred_element_type=jnp.float32) + b3_ref[...]
        t = jnp.maximum(t, 0.0)
        z_ref[...] = jnp.dot(t, w4_ref[...],
                             preferred_element_type=jnp.float32) + b4_ref[...]


def _adj_map(i):
    return (jnp.where(i <= _NB, jnp.maximum(i - 1, 0), i - 1 - _NB), 0)


def _out_map(i):
    return (jnp.clip(i - 1 - _NB, 0, _NB - 1), 0)


def _const_map(i):
    return (0, 0)


def kernel(x, Adj_, W1, b1, W2, b2, W3, b3, W4, b4):
    full = lambda r, c: pl.BlockSpec((r, c), _const_map)
    emb, z = pl.pallas_call(
        _fused_kernel,
        grid=(1 + 2 * _NB,),
        in_specs=[
            full(_N, _D),
            pl.BlockSpec((_BM, _N), _adj_map),
            full(_D, _D), full(1, _D),
            full(_D, _D), full(1, _D),
            full(_D, _D), full(1, _D),
            full(_D, _D), full(1, _D),
        ],
        out_specs=[
            pl.BlockSpec((_BM, _D), _out_map),
            pl.BlockSpec((_BM, _D), _out_map),
        ],
        out_shape=[
            jax.ShapeDtypeStruct((_N, _D), jnp.float32),
            jax.ShapeDtypeStruct((_N, _D), jnp.float32),
        ],
        scratch_shapes=[
            pltpu.VMEM((_N, _D), jnp.bfloat16),
            pltpu.VMEM((_N, _D), jnp.bfloat16),
        ],
    )(x, Adj_, W1, b1.reshape(1, _D), W2, b2.reshape(1, _D),
      W3, b3.reshape(1, _D), W4, b4.reshape(1, _D))
    return (z, emb)
